# baseline (speedup 1.0000x reference)
import jax
import jax.numpy as jnp
from jax.experimental import pallas as pl

BM = 1024
H = BM // 2
D = 2048
E = 2048


def _matmul_kernel(x1_ref, x2_ref, w_ref, b_ref, o_ref):
    dn = (((1,), (1,)), ((), ()))
    acc1 = jax.lax.dot_general(
        x1_ref[...], w_ref[...], dimension_numbers=dn,
        preferred_element_type=jnp.float32,
        precision=jax.lax.Precision.DEFAULT,
    )
    o_ref[: H, :] = acc1 + b_ref[...]
    acc2 = jax.lax.dot_general(
        x2_ref[...], w_ref[...], dimension_numbers=dn,
        preferred_element_type=jnp.float32,
        precision=jax.lax.Precision.DEFAULT,
    )
    o_ref[H :, :] = acc2 + b_ref[...]


def kernel(X, W, b):
    B, S, Din = X.shape
    M = B * S
    Xf = X.reshape(M, Din)
    b2 = b.reshape(1, E)
    grid = (M // BM,)
    out = pl.pallas_call(
        _matmul_kernel,
        grid=grid,
        in_specs=[
            pl.BlockSpec((H, Din), lambda i: (2 * i, 0)),
            pl.BlockSpec((H, Din), lambda i: (2 * i + 1, 0)),
            pl.BlockSpec((E, Din), lambda i: (0, 0)),
            pl.BlockSpec((1, E), lambda i: (0, 0)),
        ],
        out_specs=pl.BlockSpec((BM, E), lambda i: (i, 0)),
        out_shape=jax.ShapeDtypeStruct((M, E), jnp.float32),
    )(Xf, Xf, W, b2)
    return out.reshape(1, B, S, E)


# speedup vs baseline: 1.0048x; 1.0048x over previous
import jax
import jax.numpy as jnp
from jax.experimental import pallas as pl

BM = 1024
D = 2048
E = 2048


def _matmul_kernel(x_ref, w_ref, b_ref, o_ref):
    acc = jax.lax.dot_general(
        x_ref[...], w_ref[...],
        dimension_numbers=(((1,), (1,)), ((), ())),
        preferred_element_type=jnp.float32,
        precision=jax.lax.Precision.DEFAULT,
    )
    o_ref[...] = acc + b_ref[...]


def kernel(X, W, b):
    B, S, Din = X.shape
    M = B * S
    Xf = X.reshape(M, Din)
    b2 = b.reshape(1, E)
    grid = (M // BM,)
    out = pl.pallas_call(
        _matmul_kernel,
        grid=grid,
        in_specs=[
            pl.BlockSpec((BM, Din), lambda i: (i, 0)),
            pl.BlockSpec((E, Din), lambda i: (0, 0)),
            pl.BlockSpec((1, E), lambda i: (0, 0)),
        ],
        out_specs=pl.BlockSpec((BM, E), lambda i: (i, 0)),
        out_shape=jax.ShapeDtypeStruct((M, E), jnp.float32),
    )(Xf, W, b2)
    return out.reshape(1, B, S, E)
